# baseline (speedup 1.0000x reference)
import jax
import jax.numpy as jnp
from jax import lax
from jax.experimental import pallas as pl
from jax.experimental.pallas import tpu as pltpu
from jax.experimental.pallas import tpu_sc as plsc

VOCAB = 1000000
EMBED_DIM = 64
BATCH = 4096
HIST = 200

_INFO = plsc.get_sparse_core_info()
NC, NS, L = _INFO.num_cores, _INFO.num_subcores, _INFO.num_lanes
NW = NC * NS

B = BATCH * HIST
B_PER_W = B // NW
BATCH_PER_W = BATCH // NW
N_CHUNKS = BATCH_PER_W
NBUF = 4
LOOKAHEAD = 2

N_FULL_BLOCKS = VOCAB // 128
TAIL_ROWS = VOCAB - N_FULL_BLOCKS * 128
BLK_ITERS = -(-N_FULL_BLOCKS // NW)
BLK_WORDS = 128 * EMBED_DIM


def _transpose_body(tabT3, tail1d, out1d, vin0, vin1, vout0, vout1,
                    si0, si1, so0, so1):
    vins = (vin0, vin1)
    vouts = (vout0, vout1)
    isems = (si0, si1)
    osems = (so0, so1)
    wid = lax.axis_index("s") * NC + lax.axis_index("c")

    @pl.when(wid == NW - 1)
    def _():
        pltpu.sync_copy(tail1d,
                        out1d.at[pl.ds(N_FULL_BLOCKS * BLK_WORDS,
                                       TAIL_ROWS * EMBED_DIM)])

    iota16 = lax.iota(jnp.int32, 16)
    scat_base = [(iota16 + 16 * lv) * EMBED_DIM for lv in range(8)]

    def jj_of(i):
        return jnp.minimum(i * NW + wid, N_FULL_BLOCKS - 1)

    def fire_in(i, b):
        pltpu.async_copy(tabT3.at[:, :, pl.ds(jj_of(i) * 128, 128)],
                         vins[b], isems[b])

    def wait_in(i, b):
        pltpu.make_async_copy(tabT3.at[:, :, pl.ds(jj_of(i) * 128, 128)],
                              vins[b], isems[b]).wait()

    def fire_out(i, b):
        pltpu.async_copy(vouts[b],
                         out1d.at[pl.ds(jj_of(i) * BLK_WORDS, BLK_WORDS)],
                         osems[b])

    def wait_out(i, b):
        pltpu.make_async_copy(vouts[b],
                              out1d.at[pl.ds(jj_of(i) * BLK_WORDS, BLK_WORDS)],
                              osems[b]).wait()

    def transpose(b):
        vin, vout = vins[b], vouts[b]
        for i in range(8):
            for s in range(8):
                d = 8 * i + s
                for lv in range(8):
                    vals = vin[i, s, pl.ds(16 * lv, 16)]
                    plsc.store_scatter(vout, [scat_base[lv] + d], vals)

    fire_in(0, 0)

    def pair(p, carry):
        for b in range(2):
            i = p * 2 + b

            @pl.when(i + 1 < BLK_ITERS)
            def _():
                fire_in(i + 1, 1 - b)
            wait_in(i, b)

            @pl.when(i >= 2)
            def _():
                wait_out(i - 2, b)
            transpose(b)
            fire_out(i, b)
        return carry

    lax.fori_loop(0, BLK_ITERS // 2, pair, 0)
    i_last = BLK_ITERS - 1
    wait_in(i_last, 0)
    wait_out(i_last - 2, 0)
    transpose(0)
    fire_out(i_last, 0)
    wait_out(i_last - 1, 1)
    wait_out(i_last, 0)


def _gather_body(x_hbm, table_hbm, out_hbm, idx_v, rows_v,
                 g0, g1, g2, g3, o0, o1, o2, o3):
    gsems = (g0, g1, g2, g3)
    osems = (o0, o1, o2, o3)
    wid = lax.axis_index("s") * NC + lax.axis_index("c")
    base = wid * B_PER_W
    b_base = wid * BATCH_PER_W
    pltpu.sync_copy(x_hbm.at[pl.ds(base, B_PER_W)], idx_v)

    def fire_gather(c, b):
        pltpu.async_copy(
            table_hbm.at[idx_v.at[pl.ds(c * HIST, HIST)]],
            rows_v.at[b], gsems[b])

    def wait_gather(c, b):
        pltpu.make_async_copy(
            table_hbm.at[idx_v.at[pl.ds(c * HIST, HIST)]],
            rows_v.at[b], gsems[b]).wait()

    def fire_out(c, b):
        pltpu.async_copy(
            rows_v.at[b],
            out_hbm.at[b_base + c, :, pl.ds(0, EMBED_DIM)], osems[b])

    def wait_out(c, b):
        pltpu.make_async_copy(
            rows_v.at[b],
            out_hbm.at[b_base + c, :, pl.ds(0, EMBED_DIM)], osems[b]).wait()

    for c in range(LOOKAHEAD):
        fire_gather(c, c % NBUF)

    def round_body(r, carry):
        for b in range(NBUF):
            c = r * NBUF + b
            c2 = c + LOOKAHEAD
            b2 = (b + LOOKAHEAD) % NBUF

            @pl.when(c2 < N_CHUNKS)
            def _():
                @pl.when(c2 >= NBUF)
                def _():
                    wait_out(c2 - NBUF, b2)
                fire_gather(c2, b2)

            wait_gather(c, b)
            fire_out(c, b)
        return carry

    lax.fori_loop(0, N_CHUNKS // NBUF, round_body, 0)
    for k in range(NBUF):
        c = N_CHUNKS - NBUF + k
        wait_out(c, c % NBUF)


def kernel(x, table):
    mesh = plsc.VectorSubcoreMesh(core_axis_name="c", subcore_axis_name="s")

    tabT3 = table.T.reshape(8, 8, VOCAB)
    tail1d = table[VOCAB - TAIL_ROWS:, :].reshape(TAIL_ROWS * EMBED_DIM)
    flat = pl.kernel(
        _transpose_body,
        mesh=mesh,
        out_type=jax.ShapeDtypeStruct((VOCAB * EMBED_DIM,), jnp.float32),
        scratch_types=[
            pltpu.VMEM((8, 8, 128), jnp.float32),
            pltpu.VMEM((8, 8, 128), jnp.float32),
            pltpu.VMEM((BLK_WORDS,), jnp.float32),
            pltpu.VMEM((BLK_WORDS,), jnp.float32),
        ] + [pltpu.SemaphoreType.DMA] * 4,
        compiler_params=pltpu.CompilerParams(use_tc_tiling_on_sc=True,
                                             needs_layout_passes=False),
    )(tabT3, tail1d)
    table_lin = flat.reshape(VOCAB, EMBED_DIM)

    x1 = x.reshape(B).astype(jnp.int32)
    padded = pl.kernel(
        _gather_body,
        mesh=mesh,
        out_type=jax.ShapeDtypeStruct((BATCH, HIST, 2 * EMBED_DIM), jnp.float32),
        scratch_types=[
            pltpu.VMEM((B_PER_W,), jnp.int32),
            pltpu.VMEM((NBUF, HIST, EMBED_DIM), jnp.float32),
        ] + [pltpu.SemaphoreType.DMA] * (2 * NBUF),
        compiler_params=pltpu.CompilerParams(use_tc_tiling_on_sc=False),
    )(x1, table_lin)
    return padded[:, :, :EMBED_DIM]


# speedup vs baseline: 1.7005x; 1.7005x over previous
import jax
import jax.numpy as jnp
from jax import lax
from jax.experimental import pallas as pl
from jax.experimental.pallas import tpu as pltpu
from jax.experimental.pallas import tpu_sc as plsc

VOCAB = 1000000
EMBED_DIM = 64
BATCH = 4096
HIST = 200

_INFO = plsc.get_sparse_core_info()
NC, NS, L = _INFO.num_cores, _INFO.num_subcores, _INFO.num_lanes
NW = NC * NS

B = BATCH * HIST
B_PER_W = B // NW
BATCH_PER_W = BATCH // NW
N_CHUNKS = BATCH_PER_W
NBUF = 4
LOOKAHEAD = 2


def _gather_body(x_hbm, table_hbm, out_hbm, idx_v, rows_v,
                 g0, g1, g2, g3, o0, o1, o2, o3):
    gsems = (g0, g1, g2, g3)
    osems = (o0, o1, o2, o3)
    wid = lax.axis_index("s") * NC + lax.axis_index("c")
    base = wid * B_PER_W
    b_base = wid * BATCH_PER_W
    pltpu.sync_copy(x_hbm.at[pl.ds(base, B_PER_W)], idx_v)

    def fire_gather(c, b):
        pltpu.async_copy(
            table_hbm.at[idx_v.at[pl.ds(c * HIST, HIST)]],
            rows_v.at[b], gsems[b])

    def wait_gather(c, b):
        pltpu.make_async_copy(
            table_hbm.at[idx_v.at[pl.ds(c * HIST, HIST)]],
            rows_v.at[b], gsems[b]).wait()

    def fire_out(c, b):
        pltpu.async_copy(
            rows_v.at[b],
            out_hbm.at[b_base + c, :, pl.ds(0, EMBED_DIM)], osems[b])

    def wait_out(c, b):
        pltpu.make_async_copy(
            rows_v.at[b],
            out_hbm.at[b_base + c, :, pl.ds(0, EMBED_DIM)], osems[b]).wait()

    for c in range(LOOKAHEAD):
        fire_gather(c, c % NBUF)

    def round_body(r, carry):
        for b in range(NBUF):
            c = r * NBUF + b
            c2 = c + LOOKAHEAD
            b2 = (b + LOOKAHEAD) % NBUF

            @pl.when(c2 < N_CHUNKS)
            def _():
                @pl.when(c2 >= NBUF)
                def _():
                    wait_out(c2 - NBUF, b2)
                fire_gather(c2, b2)

            wait_gather(c, b)
            fire_out(c, b)
        return carry

    lax.fori_loop(0, N_CHUNKS // NBUF, round_body, 0)

    for k in range(NBUF):
        c = N_CHUNKS - NBUF + k
        wait_out(c, c % NBUF)


def kernel(x, table):
    x1 = x.reshape(B).astype(jnp.int32)
    mesh = plsc.VectorSubcoreMesh(core_axis_name="c", subcore_axis_name="s")
    padded = pl.kernel(
        _gather_body,
        mesh=mesh,
        out_type=jax.ShapeDtypeStruct((BATCH, HIST, 2 * EMBED_DIM), jnp.float32),
        scratch_types=[
            pltpu.VMEM((B_PER_W,), jnp.int32),
            pltpu.VMEM((NBUF, HIST, EMBED_DIM), jnp.float32),
        ] + [pltpu.SemaphoreType.DMA] * (2 * NBUF),
        compiler_params=pltpu.CompilerParams(use_tc_tiling_on_sc=False),
    )(x1, table)
    return padded[:, :, :EMBED_DIM]
